# baseline (speedup 1.0000x reference)
import functools

import jax
import jax.numpy as jnp
from jax import lax
from jax.experimental import pallas as pl
from jax.experimental.pallas import tpu as pltpu
from jax.experimental.pallas import tpu_sc as plsc

NC = 2
NS = 16
NW = NC * NS
L = 16

K = 7
D = 128
CH = 16
IDXW = CH * K
NBUF = 6
OBUF = 2
CHUNKS = 84
WPN = CH * CHUNKS
PAD_NODES = NW * WPN
SCALE = 1.0 / K


def _pool_body(nn, tail, x_hbm, idx_hbm, out_hbm, idx_v, r0, r1, r2, r3,
               r4, r5, ob0, ob1, g0, g1, g2, g3, g4, g5, s0, s1):
    rows = [r0, r1, r2, r3, r4, r5]
    gsem = [g0, g1, g2, g3, g4, g5]
    obs = [ob0, ob1]
    ssem = [s0, s1]
    w = lax.axis_index("s") * NC + lax.axis_index("c")

    pltpu.sync_copy(idx_hbm.at[pl.ds(w * CHUNKS * IDXW, CHUNKS * IDXW)],
                    idx_v)

    def start_gather(i, b):
        pltpu.make_async_copy(x_hbm.at[idx_v.at[pl.ds(i * IDXW, IDXW)]],
                              rows[b], gsem[b]).start()

    def wait_gather(b):
        pltpu.make_async_copy(x_hbm.at[pl.ds(0, IDXW)], rows[b],
                              gsem[b]).wait()

    def start_store(base, b):
        pltpu.make_async_copy(obs[b], out_hbm.at[pl.ds(base, CH)],
                              ssem[b]).start()

    def wait_store(b):
        pltpu.make_async_copy(obs[b], out_hbm.at[pl.ds(0, CH)],
                              ssem[b]).wait()

    def reduce(b, ob):
        src = rows[b]
        dst = obs[ob]

        @plsc.parallel_loop(0, CH, 1, unroll=4)
        def _(n):
            rbase = n * K
            for g in range(D // L):
                sl = pl.ds(g * L, L)
                acc = src[rbase, sl]
                for j in range(1, K):
                    acc = acc + src[rbase + j, sl]
                dst[n, sl] = acc * SCALE

    for b in range(NBUF):
        start_gather(b, b)

    def outer(t, carry):
        for b in range(NBUF):
            i = t * NBUF + b
            wait_gather(b)
            base_raw = w * WPN + i * CH
            rem = nn - base_raw

            @pl.when(jnp.logical_and(i >= OBUF, nn - (base_raw - OBUF * CH) >= CH))
            def _():
                wait_store(b % OBUF)

            reduce(b, b % OBUF)

            @pl.when(i + NBUF < CHUNKS)
            def _():
                start_gather(i + NBUF, b)

            @pl.when(rem >= CH)
            def _():
                start_store(base_raw, b % OBUF)

            @pl.when(jnp.logical_and(rem > 0, rem < CH))
            def _():
                pltpu.sync_copy(obs[b % OBUF].at[pl.ds(0, tail)],
                                out_hbm.at[pl.ds(base_raw, tail)])
        return carry

    lax.fori_loop(0, CHUNKS // NBUF, outer, 0)

    for e in range(OBUF):
        i = CHUNKS - OBUF + e

        @pl.when(nn - (w * WPN + i * CH) >= CH)
        def _():
            wait_store(i % OBUF)


def kernel(x, neigh_orders):
    nn = (x.shape[0] + 6) // 4
    pad = jnp.arange(PAD_NODES * K - nn * K, dtype=jnp.int32) % x.shape[0]
    idx = jnp.concatenate([neigh_orders[: nn * K], pad])

    mesh = plsc.VectorSubcoreMesh(core_axis_name="c", subcore_axis_name="s")
    pool = pl.kernel(
        functools.partial(_pool_body, nn, nn % CH),
        mesh=mesh,
        out_type=jax.ShapeDtypeStruct((nn, D), jnp.float32),
        scratch_types=[
            pltpu.VMEM((CHUNKS * IDXW,), jnp.int32),
            pltpu.VMEM((IDXW, D), jnp.float32),
            pltpu.VMEM((IDXW, D), jnp.float32),
            pltpu.VMEM((IDXW, D), jnp.float32),
            pltpu.VMEM((IDXW, D), jnp.float32),
            pltpu.VMEM((IDXW, D), jnp.float32),
            pltpu.VMEM((IDXW, D), jnp.float32),
            pltpu.VMEM((CH, D), jnp.float32),
            pltpu.VMEM((CH, D), jnp.float32),
            pltpu.SemaphoreType.DMA,
            pltpu.SemaphoreType.DMA,
            pltpu.SemaphoreType.DMA,
            pltpu.SemaphoreType.DMA,
            pltpu.SemaphoreType.DMA,
            pltpu.SemaphoreType.DMA,
            pltpu.SemaphoreType.DMA,
            pltpu.SemaphoreType.DMA,
        ],
    )
    return pool(x, idx)


# speedup vs baseline: 1.1101x; 1.1101x over previous
import functools

import jax
import jax.numpy as jnp
from jax import lax
from jax.experimental import pallas as pl
from jax.experimental.pallas import tpu as pltpu
from jax.experimental.pallas import tpu_sc as plsc

NC = 2
NS = 16
NW = NC * NS
L = 16

K = 7
D = 128
CH = 16
IDXW = CH * K
NBUF = 4
OBUF = 2
CHUNKS = 84
WPN = CH * CHUNKS
PAD_NODES = NW * WPN
SCALE = 1.0 / K


def _pool_body(nn, tail, x_hbm, idx_hbm, out_hbm, idx_v, r0, r1, r2, r3,
               ob0, ob1, g0, g1, g2, g3, s0, s1):
    rows = [r0, r1, r2, r3]
    gsem = [g0, g1, g2, g3]
    obs = [ob0, ob1]
    ssem = [s0, s1]
    w = lax.axis_index("s") * NC + lax.axis_index("c")

    pltpu.sync_copy(idx_hbm.at[pl.ds(w * CHUNKS * IDXW, CHUNKS * IDXW)],
                    idx_v)

    def start_gather(i, b):
        pltpu.make_async_copy(x_hbm.at[idx_v.at[pl.ds(i * IDXW, IDXW)]],
                              rows[b], gsem[b]).start()

    def wait_gather(b):
        pltpu.make_async_copy(x_hbm.at[pl.ds(0, IDXW)], rows[b],
                              gsem[b]).wait()

    def start_store(base, b):
        pltpu.make_async_copy(obs[b], out_hbm.at[pl.ds(base, CH)],
                              ssem[b]).start()

    def wait_store(b):
        pltpu.make_async_copy(obs[b], out_hbm.at[pl.ds(0, CH)],
                              ssem[b]).wait()

    def reduce(b, ob):
        src = rows[b]
        dst = obs[ob]

        @plsc.parallel_loop(0, CH, 1, unroll=4)
        def _(n):
            rbase = n * K
            for g in range(D // L):
                sl = pl.ds(g * L, L)
                acc = src[rbase, sl]
                for j in range(1, K):
                    acc = acc + src[rbase + j, sl]
                dst[n, sl] = acc * SCALE

    for b in range(NBUF):
        start_gather(b, b)

    def outer(t, carry):
        for b in range(NBUF):
            i = t * NBUF + b
            wait_gather(b)
            base_raw = w * WPN + i * CH
            rem = nn - base_raw

            @pl.when(jnp.logical_and(i >= OBUF, nn - (base_raw - OBUF * CH) >= CH))
            def _():
                wait_store(b % OBUF)

            reduce(b, b % OBUF)

            @pl.when(i + NBUF < CHUNKS)
            def _():
                start_gather(i + NBUF, b)

            @pl.when(rem >= CH)
            def _():
                start_store(base_raw, b % OBUF)

            @pl.when(jnp.logical_and(rem > 0, rem < CH))
            def _():
                pltpu.sync_copy(obs[b % OBUF].at[pl.ds(0, tail)],
                                out_hbm.at[pl.ds(base_raw, tail)])
        return carry

    lax.fori_loop(0, CHUNKS // NBUF, outer, 0)

    for e in range(OBUF):
        i = CHUNKS - OBUF + e

        @pl.when(nn - (w * WPN + i * CH) >= CH)
        def _():
            wait_store(i % OBUF)


def kernel(x, neigh_orders):
    nn = (x.shape[0] + 6) // 4
    pad = jnp.arange(PAD_NODES * K - nn * K, dtype=jnp.int32) % x.shape[0]
    idx = jnp.concatenate([neigh_orders[: nn * K], pad])

    mesh = plsc.VectorSubcoreMesh(core_axis_name="c", subcore_axis_name="s")
    pool = pl.kernel(
        functools.partial(_pool_body, nn, nn % CH),
        mesh=mesh,
        out_type=jax.ShapeDtypeStruct((nn, D), jnp.float32),
        scratch_types=[
            pltpu.VMEM((CHUNKS * IDXW,), jnp.int32),
            pltpu.VMEM((IDXW, D), jnp.float32),
            pltpu.VMEM((IDXW, D), jnp.float32),
            pltpu.VMEM((IDXW, D), jnp.float32),
            pltpu.VMEM((IDXW, D), jnp.float32),
            pltpu.VMEM((CH, D), jnp.float32),
            pltpu.VMEM((CH, D), jnp.float32),
            pltpu.SemaphoreType.DMA,
            pltpu.SemaphoreType.DMA,
            pltpu.SemaphoreType.DMA,
            pltpu.SemaphoreType.DMA,
            pltpu.SemaphoreType.DMA,
            pltpu.SemaphoreType.DMA,
        ],
    )
    return pool(x, idx)
